# speedup vs baseline: 13.4043x; 13.4043x over previous
import functools

import jax
import jax.numpy as jnp
from jax import lax
from jax.experimental import pallas as pl
from jax.experimental.pallas import tpu as pltpu
from jax.experimental.pallas import tpu_sc as plsc

VOCAB = 100000
EMBED = 128
BATCH = 4096
N_TOK = 819200

NS = 16
BLK = 128
TOK_ROWS = N_TOK // BLK
ROWS_PER_TILE = TOK_ROWS // NS
CHUNK_ROWS = 16
N_CHUNKS = ROWS_PER_TILE // CHUNK_ROWS
ROWS_OUT_PER_TILE = BATCH // NS


def _body(tok_hbm, bid_hbm, w_hbm, zeros_hbm, out_hbm,
          tok_v, bid_v, rows_v, acc_sh, gsem):
    c = lax.axis_index("c")
    s = lax.axis_index("s")

    pltpu.sync_copy(zeros_hbm, acc_sh.at[pl.ds(s * ROWS_OUT_PER_TILE,
                                               ROWS_OUT_PER_TILE)])
    plsc.subcore_barrier()

    row0 = s * ROWS_PER_TILE

    def chunk(ci, carry):
        base = row0 + ci * CHUNK_ROWS
        pltpu.sync_copy(tok_hbm.at[c, pl.ds(base, CHUNK_ROWS)], tok_v)
        pltpu.sync_copy(bid_hbm.at[c, pl.ds(base, CHUNK_ROWS)], bid_v)

        def blk(b, carry2):
            pltpu.async_copy(w_hbm.at[tok_v.at[b]], rows_v, gsem).wait()
            pltpu.sync_copy(rows_v, acc_sh.at[bid_v.at[b]], add=True)
            return carry2

        lax.fori_loop(0, CHUNK_ROWS, blk, 0, unroll=False)
        return carry

    lax.fori_loop(0, N_CHUNKS, chunk, 0, unroll=False)
    plsc.subcore_barrier()

    o = s * ROWS_OUT_PER_TILE
    pltpu.sync_copy(acc_sh.at[pl.ds(o, ROWS_OUT_PER_TILE)],
                    out_hbm.at[pl.ds(o, ROWS_OUT_PER_TILE),
                               pl.ds(c * EMBED, EMBED)])


@jax.jit
def kernel(added_tokens, added_batch_ids, deled_tokens, deled_batch_ids, W):
    tok = jnp.stack([added_tokens.astype(jnp.int32),
                     deled_tokens.astype(jnp.int32)]).reshape(2, TOK_ROWS, BLK)
    bid = jnp.stack([added_batch_ids.astype(jnp.int32),
                     deled_batch_ids.astype(jnp.int32)]).reshape(2, TOK_ROWS, BLK)
    zeros = jnp.zeros((ROWS_OUT_PER_TILE, EMBED), jnp.float32)

    mesh = plsc.VectorSubcoreMesh(core_axis_name="c", subcore_axis_name="s")
    run = pl.kernel(
        _body,
        out_type=jax.ShapeDtypeStruct((BATCH, 2 * EMBED), jnp.float32),
        mesh=mesh,
        scratch_types=[
            pltpu.VMEM((CHUNK_ROWS, BLK), jnp.int32),
            pltpu.VMEM((CHUNK_ROWS, BLK), jnp.int32),
            pltpu.VMEM((BLK, EMBED), jnp.float32),
            pltpu.VMEM_SHARED((BATCH, EMBED), jnp.float32),
            pltpu.SemaphoreType.DMA,
        ],
    )
    return run(tok, bid, W, zeros)


# speedup vs baseline: 21.4006x; 1.5965x over previous
import functools

import jax
import jax.numpy as jnp
from jax import lax
from jax.experimental import pallas as pl
from jax.experimental.pallas import tpu as pltpu
from jax.experimental.pallas import tpu_sc as plsc

VOCAB = 100000
EMBED = 128
BATCH = 4096
N_TOK = 819200

NS = 16
BLK = 128
TOK_ROWS = N_TOK // BLK
ROWS_PER_TILE = TOK_ROWS // NS
CHUNK_ROWS = 16
N_CHUNKS = ROWS_PER_TILE // CHUNK_ROWS
NBUF = 4
GROUPS = CHUNK_ROWS // NBUF
ROWS_OUT_PER_TILE = BATCH // NS


def _body(tok_hbm, bid_hbm, w_hbm, zeros_hbm, out_hbm,
          tok_v, bid_v, rows_v, acc_sh, gsem, tsem, bsem):
    c = lax.axis_index("c")
    s = lax.axis_index("s")

    pltpu.sync_copy(zeros_hbm, acc_sh.at[pl.ds(s * ROWS_OUT_PER_TILE,
                                               ROWS_OUT_PER_TILE)])
    plsc.subcore_barrier()

    row0 = s * ROWS_PER_TILE

    pltpu.sync_copy(tok_hbm.at[c, pl.ds(row0, CHUNK_ROWS)], tok_v.at[0])
    pltpu.sync_copy(bid_hbm.at[c, pl.ds(row0, CHUNK_ROWS)], bid_v.at[0])

    def chunk(ci, carry):
        p = lax.rem(ci, 2)
        base = row0 + ci * CHUNK_ROWS

        @pl.when(ci > 0)
        def _():
            pltpu.make_async_copy(tok_hbm.at[c, pl.ds(base, CHUNK_ROWS)],
                                  tok_v.at[p], tsem).wait()
            pltpu.make_async_copy(bid_hbm.at[c, pl.ds(base, CHUNK_ROWS)],
                                  bid_v.at[p], bsem).wait()

        @pl.when(ci + 1 < N_CHUNKS)
        def _():
            pn = lax.rem(ci + 1, 2)
            nbase = base + CHUNK_ROWS
            pltpu.async_copy(tok_hbm.at[c, pl.ds(nbase, CHUNK_ROWS)],
                             tok_v.at[pn], tsem)
            pltpu.async_copy(bid_hbm.at[c, pl.ds(nbase, CHUNK_ROWS)],
                             bid_v.at[pn], bsem)

        for k in range(NBUF):
            pltpu.async_copy(w_hbm.at[tok_v.at[p, k]], rows_v.at[k],
                             gsem.at[k])

        def group(g, carry2):
            for k in range(NBUF):
                b = g * NBUF + k
                pltpu.make_async_copy(w_hbm.at[tok_v.at[p, b]],
                                      rows_v.at[k], gsem.at[k]).wait()
                pltpu.sync_copy(rows_v.at[k], acc_sh.at[bid_v.at[p, b]],
                                add=True)

                @pl.when(g < GROUPS - 1)
                def _():
                    pltpu.async_copy(w_hbm.at[tok_v.at[p, b + NBUF]],
                                     rows_v.at[k], gsem.at[k])
            return carry2

        lax.fori_loop(0, GROUPS, group, 0, unroll=False)
        return carry

    lax.fori_loop(0, N_CHUNKS, chunk, 0, unroll=False)
    plsc.subcore_barrier()

    o = s * ROWS_OUT_PER_TILE
    pltpu.sync_copy(acc_sh.at[pl.ds(o, ROWS_OUT_PER_TILE)],
                    out_hbm.at[pl.ds(o, ROWS_OUT_PER_TILE),
                               pl.ds(c * EMBED, EMBED)])


@jax.jit
def kernel(added_tokens, added_batch_ids, deled_tokens, deled_batch_ids, W):
    tok = jnp.stack([added_tokens.astype(jnp.int32),
                     deled_tokens.astype(jnp.int32)]).reshape(2, TOK_ROWS, BLK)
    bid = jnp.stack([added_batch_ids.astype(jnp.int32),
                     deled_batch_ids.astype(jnp.int32)]).reshape(2, TOK_ROWS, BLK)
    zeros = jnp.zeros((ROWS_OUT_PER_TILE, EMBED), jnp.float32)

    mesh = plsc.VectorSubcoreMesh(core_axis_name="c", subcore_axis_name="s")
    run = pl.kernel(
        _body,
        out_type=jax.ShapeDtypeStruct((BATCH, 2 * EMBED), jnp.float32),
        mesh=mesh,
        scratch_types=[
            pltpu.VMEM((2, CHUNK_ROWS, BLK), jnp.int32),
            pltpu.VMEM((2, CHUNK_ROWS, BLK), jnp.int32),
            pltpu.VMEM((NBUF, BLK, EMBED), jnp.float32),
            pltpu.VMEM_SHARED((BATCH, EMBED), jnp.float32),
            pltpu.SemaphoreType.DMA((NBUF,)),
            pltpu.SemaphoreType.DMA,
            pltpu.SemaphoreType.DMA,
        ],
    )
    return run(tok, bid, W, zeros)


# speedup vs baseline: 24.0724x; 1.1249x over previous
import jax
import jax.numpy as jnp
from jax import lax
from jax.experimental import pallas as pl
from jax.experimental.pallas import tpu as pltpu
from jax.experimental.pallas import tpu_sc as plsc

VOCAB = 100000
EMBED = 128
BATCH = 4096
N_TOK = 819200

NS = 16
BLK = 128
TOK_ROWS = N_TOK // BLK
ROWS_PER_TILE = TOK_ROWS // NS
CHUNK_ROWS = 16
N_CHUNKS = ROWS_PER_TILE // CHUNK_ROWS
NBUF = 5
LOOK = 3
GROUPS = ROWS_PER_TILE // NBUF
ROWS_OUT_PER_TILE = BATCH // NS


def _body(tok_hbm, bid_hbm, w_hbm, zeros_hbm, out_hbm,
          tok_v, bid_v, rows_v, acc_sh, gsem, ssem, tsem, bsem):
    c = lax.axis_index("c")
    s = lax.axis_index("s")

    pltpu.sync_copy(zeros_hbm, acc_sh.at[pl.ds(s * ROWS_OUT_PER_TILE,
                                               ROWS_OUT_PER_TILE)])
    plsc.subcore_barrier()

    row0 = s * ROWS_PER_TILE

    pltpu.sync_copy(tok_hbm.at[c, pl.ds(row0, CHUNK_ROWS)], tok_v.at[0])
    pltpu.sync_copy(bid_hbm.at[c, pl.ds(row0, CHUNK_ROWS)], bid_v.at[0])
    pltpu.async_copy(tok_hbm.at[c, pl.ds(row0 + CHUNK_ROWS, CHUNK_ROWS)],
                     tok_v.at[1], tsem)
    pltpu.async_copy(bid_hbm.at[c, pl.ds(row0 + CHUNK_ROWS, CHUNK_ROWS)],
                     bid_v.at[1], bsem)

    for k in range(LOOK):
        pltpu.async_copy(w_hbm.at[tok_v.at[0, k]], rows_v.at[k], gsem.at[k])

    def group(g, carry):
        for k in range(NBUF):
            b = g * NBUF + k
            p = lax.rem(lax.div(b, CHUNK_ROWS), 3)
            r = lax.rem(b, CHUNK_ROWS)

            pltpu.make_async_copy(w_hbm.at[tok_v.at[p, r]],
                                  rows_v.at[k], gsem.at[k]).wait()
            pltpu.async_copy(rows_v.at[k], acc_sh.at[bid_v.at[p, r]],
                             ssem.at[k], add=True)

            j = b + LOOK
            cj = lax.div(j, CHUNK_ROWS)
            pj = lax.rem(cj, 3)
            rj = lax.rem(j, CHUNK_ROWS)
            kj = (k + LOOK) % NBUF

            @pl.when(jnp.logical_and(rj == 0, j < ROWS_PER_TILE))
            def _():
                base = row0 + cj * CHUNK_ROWS
                pltpu.make_async_copy(tok_hbm.at[c, pl.ds(base, CHUNK_ROWS)],
                                      tok_v.at[pj], tsem).wait()
                pltpu.make_async_copy(bid_hbm.at[c, pl.ds(base, CHUNK_ROWS)],
                                      bid_v.at[pj], bsem).wait()

                @pl.when(cj + 1 < N_CHUNKS)
                def _():
                    nbase = base + CHUNK_ROWS
                    pn = lax.rem(cj + 1, 3)
                    pltpu.async_copy(
                        tok_hbm.at[c, pl.ds(nbase, CHUNK_ROWS)],
                        tok_v.at[pn], tsem)
                    pltpu.async_copy(
                        bid_hbm.at[c, pl.ds(nbase, CHUNK_ROWS)],
                        bid_v.at[pn], bsem)

            @pl.when(j < ROWS_PER_TILE)
            def _():
                @pl.when(b >= NBUF - LOOK)
                def _():
                    pltpu.make_async_copy(rows_v.at[kj],
                                          acc_sh.at[bid_v.at[pj, rj]],
                                          ssem.at[kj]).wait()
                pltpu.async_copy(w_hbm.at[tok_v.at[pj, rj]],
                                 rows_v.at[kj], gsem.at[kj])
        return carry

    lax.fori_loop(0, GROUPS, group, 0, unroll=False)

    for k in range(NBUF):
        pltpu.make_async_copy(rows_v.at[k], acc_sh.at[bid_v.at[0, 0]],
                              ssem.at[k]).wait()
    plsc.subcore_barrier()

    o = s * ROWS_OUT_PER_TILE
    pltpu.sync_copy(acc_sh.at[pl.ds(o, ROWS_OUT_PER_TILE)],
                    out_hbm.at[pl.ds(o, ROWS_OUT_PER_TILE),
                               pl.ds(c * EMBED, EMBED)])


@jax.jit
def kernel(added_tokens, added_batch_ids, deled_tokens, deled_batch_ids, W):
    tok = jnp.stack([added_tokens.astype(jnp.int32),
                     deled_tokens.astype(jnp.int32)]).reshape(2, TOK_ROWS, BLK)
    bid = jnp.stack([added_batch_ids.astype(jnp.int32),
                     deled_batch_ids.astype(jnp.int32)]).reshape(2, TOK_ROWS, BLK)
    zeros = jnp.zeros((ROWS_OUT_PER_TILE, EMBED), jnp.float32)

    mesh = plsc.VectorSubcoreMesh(core_axis_name="c", subcore_axis_name="s")
    run = pl.kernel(
        _body,
        out_type=jax.ShapeDtypeStruct((BATCH, 2 * EMBED), jnp.float32),
        mesh=mesh,
        scratch_types=[
            pltpu.VMEM((3, CHUNK_ROWS, BLK), jnp.int32),
            pltpu.VMEM((3, CHUNK_ROWS, BLK), jnp.int32),
            pltpu.VMEM((NBUF, BLK, EMBED), jnp.float32),
            pltpu.VMEM_SHARED((BATCH, EMBED), jnp.float32),
            pltpu.SemaphoreType.DMA((NBUF,)),
            pltpu.SemaphoreType.DMA((NBUF,)),
            pltpu.SemaphoreType.DMA,
            pltpu.SemaphoreType.DMA,
        ],
    )
    return run(tok, bid, W, zeros)


# speedup vs baseline: 24.2119x; 1.0058x over previous
import jax
import jax.numpy as jnp
from jax import lax
from jax.experimental import pallas as pl
from jax.experimental.pallas import tpu as pltpu
from jax.experimental.pallas import tpu_sc as plsc

VOCAB = 100000
EMBED = 128
BATCH = 4096
N_TOK = 819200

NS = 16
BLK = 128
TOK_ROWS = N_TOK // BLK
ROWS_PER_TILE = TOK_ROWS // NS
CHUNK_ROWS = 16
N_CHUNKS = ROWS_PER_TILE // CHUNK_ROWS
NBUF = 5
LOOK = 4
GROUPS = ROWS_PER_TILE // NBUF
ROWS_OUT_PER_TILE = BATCH // NS


def _body(tok_hbm, bid_hbm, w_hbm, zeros_hbm, out_hbm,
          tok_v, bid_v, rows_v, acc_sh, gsem, ssem, tsem, bsem):
    c = lax.axis_index("c")
    s = lax.axis_index("s")

    row0 = s * ROWS_PER_TILE

    pltpu.sync_copy(tok_hbm.at[c, pl.ds(row0, CHUNK_ROWS)], tok_v.at[0])
    pltpu.sync_copy(bid_hbm.at[c, pl.ds(row0, CHUNK_ROWS)], bid_v.at[0])
    pltpu.async_copy(tok_hbm.at[c, pl.ds(row0 + CHUNK_ROWS, CHUNK_ROWS)],
                     tok_v.at[1], tsem)
    pltpu.async_copy(bid_hbm.at[c, pl.ds(row0 + CHUNK_ROWS, CHUNK_ROWS)],
                     bid_v.at[1], bsem)

    for k in range(LOOK):
        pltpu.async_copy(w_hbm.at[tok_v.at[0, k]], rows_v.at[k], gsem.at[k])

    pltpu.sync_copy(zeros_hbm, acc_sh.at[pl.ds(s * ROWS_OUT_PER_TILE,
                                               ROWS_OUT_PER_TILE)])
    plsc.subcore_barrier()

    def group(g, carry):
        for k in range(NBUF):
            b = g * NBUF + k
            p = lax.rem(lax.div(b, CHUNK_ROWS), 3)
            r = lax.rem(b, CHUNK_ROWS)

            pltpu.make_async_copy(w_hbm.at[tok_v.at[p, r]],
                                  rows_v.at[k], gsem.at[k]).wait()
            pltpu.async_copy(rows_v.at[k], acc_sh.at[bid_v.at[p, r]],
                             ssem.at[k], add=True)

            j = b + LOOK
            cj = lax.div(j, CHUNK_ROWS)
            pj = lax.rem(cj, 3)
            rj = lax.rem(j, CHUNK_ROWS)
            kj = (k + LOOK) % NBUF

            @pl.when(jnp.logical_and(rj == 0, j < ROWS_PER_TILE))
            def _():
                base = row0 + cj * CHUNK_ROWS
                pltpu.make_async_copy(tok_hbm.at[c, pl.ds(base, CHUNK_ROWS)],
                                      tok_v.at[pj], tsem).wait()
                pltpu.make_async_copy(bid_hbm.at[c, pl.ds(base, CHUNK_ROWS)],
                                      bid_v.at[pj], bsem).wait()

                @pl.when(cj + 1 < N_CHUNKS)
                def _():
                    nbase = base + CHUNK_ROWS
                    pn = lax.rem(cj + 1, 3)
                    pltpu.async_copy(
                        tok_hbm.at[c, pl.ds(nbase, CHUNK_ROWS)],
                        tok_v.at[pn], tsem)
                    pltpu.async_copy(
                        bid_hbm.at[c, pl.ds(nbase, CHUNK_ROWS)],
                        bid_v.at[pn], bsem)

            @pl.when(j < ROWS_PER_TILE)
            def _():
                @pl.when(b >= NBUF - LOOK)
                def _():
                    pltpu.make_async_copy(rows_v.at[kj],
                                          acc_sh.at[bid_v.at[pj, rj]],
                                          ssem.at[kj]).wait()
                pltpu.async_copy(w_hbm.at[tok_v.at[pj, rj]],
                                 rows_v.at[kj], gsem.at[kj])
        return carry

    lax.fori_loop(0, GROUPS, group, 0, unroll=False)

    for k in range(NBUF):
        pltpu.make_async_copy(rows_v.at[k], acc_sh.at[bid_v.at[0, 0]],
                              ssem.at[k]).wait()
    plsc.subcore_barrier()

    o = s * ROWS_OUT_PER_TILE
    pltpu.sync_copy(acc_sh.at[pl.ds(o, ROWS_OUT_PER_TILE)],
                    out_hbm.at[pl.ds(o, ROWS_OUT_PER_TILE),
                               pl.ds(c * EMBED, EMBED)])


@jax.jit
def kernel(added_tokens, added_batch_ids, deled_tokens, deled_batch_ids, W):
    tok = jnp.stack([added_tokens.astype(jnp.int32),
                     deled_tokens.astype(jnp.int32)]).reshape(2, TOK_ROWS, BLK)
    bid = jnp.stack([added_batch_ids.astype(jnp.int32),
                     deled_batch_ids.astype(jnp.int32)]).reshape(2, TOK_ROWS, BLK)
    zeros = jnp.zeros((ROWS_OUT_PER_TILE, EMBED), jnp.float32)

    mesh = plsc.VectorSubcoreMesh(core_axis_name="c", subcore_axis_name="s")
    run = pl.kernel(
        _body,
        out_type=jax.ShapeDtypeStruct((BATCH, 2 * EMBED), jnp.float32),
        mesh=mesh,
        scratch_types=[
            pltpu.VMEM((3, CHUNK_ROWS, BLK), jnp.int32),
            pltpu.VMEM((3, CHUNK_ROWS, BLK), jnp.int32),
            pltpu.VMEM((NBUF, BLK, EMBED), jnp.float32),
            pltpu.VMEM_SHARED((BATCH, EMBED), jnp.float32),
            pltpu.SemaphoreType.DMA((NBUF,)),
            pltpu.SemaphoreType.DMA((NBUF,)),
            pltpu.SemaphoreType.DMA,
            pltpu.SemaphoreType.DMA,
        ],
    )
    return run(tok, bid, W, zeros)


# speedup vs baseline: 27.4940x; 1.1356x over previous
import jax
import jax.numpy as jnp
from jax import lax
from jax.experimental import pallas as pl
from jax.experimental.pallas import tpu as pltpu
from jax.experimental.pallas import tpu_sc as plsc

VOCAB = 100000
EMBED = 128
BATCH = 4096
N_TOK = 819200

NS = 16
BLK = 128
LANES = 16
NCH = EMBED // LANES
TOK_ROWS = N_TOK // BLK
ROWS_PER_TILE = TOK_ROWS // NS
CHUNK_ROWS = 8
N_CHUNKS = ROWS_PER_TILE // CHUNK_ROWS
NBUF = 5
LOOK = 4
GROUPS = ROWS_PER_TILE // NBUF
ROWS_OUT_PER_TILE = BATCH // NS
FL = 16
ACC_ROWS = BATCH + FL


def _body(tok_hbm, bid_hbm, w_hbm, zeros_hbm, out_hbm,
          tok_v, bid_v, rows_v, fids_v, acc_sh,
          gsem, ssem, tsem, bsem):
    c = lax.axis_index("c")
    s = lax.axis_index("s")

    row0 = s * ROWS_PER_TILE

    pltpu.sync_copy(tok_hbm.at[c, pl.ds(row0, CHUNK_ROWS)], tok_v.at[0])
    pltpu.sync_copy(bid_hbm.at[c, pl.ds(row0, CHUNK_ROWS)], bid_v.at[0])
    pltpu.async_copy(tok_hbm.at[c, pl.ds(row0 + CHUNK_ROWS, CHUNK_ROWS)],
                     tok_v.at[1], tsem)
    pltpu.async_copy(bid_hbm.at[c, pl.ds(row0 + CHUNK_ROWS, CHUNK_ROWS)],
                     bid_v.at[1], bsem)

    for k in range(LOOK):
        pltpu.async_copy(w_hbm.at[tok_v.at[0, k]], rows_v.at[k], gsem.at[k])

    pltpu.sync_copy(zeros_hbm, acc_sh.at[pl.ds(s * ROWS_OUT_PER_TILE,
                                               ROWS_OUT_PER_TILE)])
    plsc.subcore_barrier()

    def _bid_ends(pb, rb):
        lo = bid_v[pb, rb, pl.ds(0, LANES)]
        hi = bid_v[pb, rb, pl.ds(BLK - LANES, LANES)]
        return lo[0], hi[LANES - 1]

    def _is_fast(blk):
        pb = lax.rem(lax.div(blk, CHUNK_ROWS), 3)
        rb = lax.rem(blk, CHUNK_ROWS)
        first, last = _bid_ends(pb, rb)
        return first == last

    def _drain(slot, blk):
        fast = _is_fast(blk)

        @pl.when(fast)
        def _():
            pltpu.make_async_copy(rows_v.at[slot, pl.ds(0, FL)],
                                  acc_sh.at[fids_v.at[slot]],
                                  ssem.at[slot]).wait()

        @pl.when(jnp.logical_not(fast))
        def _():
            pb = lax.rem(lax.div(blk, CHUNK_ROWS), 3)
            rb = lax.rem(blk, CHUNK_ROWS)
            pltpu.make_async_copy(rows_v.at[slot],
                                  acc_sh.at[bid_v.at[pb, rb]],
                                  ssem.at[slot]).wait()

    def group(g, carry):
        for k in range(NBUF):
            b = g * NBUF + k
            p = lax.rem(lax.div(b, CHUNK_ROWS), 3)
            r = lax.rem(b, CHUNK_ROWS)

            pltpu.make_async_copy(w_hbm.at[tok_v.at[p, r]],
                                  rows_v.at[k], gsem.at[k]).wait()

            first_id, last_id = _bid_ends(p, r)
            fast = first_id == last_id

            @pl.when(fast)
            def _():
                def red(i, accs):
                    base = 2 * i
                    a1 = tuple(accs[j] + rows_v[k, base, pl.ds(LANES * j,
                                                               LANES)]
                               for j in range(NCH))
                    return tuple(a1[j] + rows_v[k, base + 1,
                                                pl.ds(LANES * j, LANES)]
                                 for j in range(NCH))

                accs = lax.fori_loop(
                    0, BLK // 2, red,
                    tuple(jnp.zeros((LANES,), jnp.float32)
                          for _ in range(NCH)))
                for j in range(NCH):
                    rows_v[k, 0, pl.ds(LANES * j, LANES)] = accs[j]
                iota = lax.iota(jnp.int32, LANES)
                fids_v[k] = jnp.where(iota == 0, first_id, BATCH + iota)
                pltpu.async_copy(rows_v.at[k, pl.ds(0, FL)],
                                 acc_sh.at[fids_v.at[k]],
                                 ssem.at[k], add=True)

            @pl.when(jnp.logical_not(fast))
            def _():
                pltpu.async_copy(rows_v.at[k], acc_sh.at[bid_v.at[p, r]],
                                 ssem.at[k], add=True)

            j = b + LOOK
            cj = lax.div(j, CHUNK_ROWS)
            pj = lax.rem(cj, 3)
            rj = lax.rem(j, CHUNK_ROWS)
            kj = (k + LOOK) % NBUF

            @pl.when(jnp.logical_and(rj == 0, j < ROWS_PER_TILE))
            def _():
                base = row0 + cj * CHUNK_ROWS
                pltpu.make_async_copy(tok_hbm.at[c, pl.ds(base, CHUNK_ROWS)],
                                      tok_v.at[pj], tsem).wait()
                pltpu.make_async_copy(bid_hbm.at[c, pl.ds(base, CHUNK_ROWS)],
                                      bid_v.at[pj], bsem).wait()

                @pl.when(cj + 1 < N_CHUNKS)
                def _():
                    nbase = base + CHUNK_ROWS
                    pn = lax.rem(cj + 1, 3)
                    pltpu.async_copy(
                        tok_hbm.at[c, pl.ds(nbase, CHUNK_ROWS)],
                        tok_v.at[pn], tsem)
                    pltpu.async_copy(
                        bid_hbm.at[c, pl.ds(nbase, CHUNK_ROWS)],
                        bid_v.at[pn], bsem)

            @pl.when(j < ROWS_PER_TILE)
            def _():
                @pl.when(b >= NBUF - LOOK)
                def _():
                    _drain(kj, j - NBUF)
                pltpu.async_copy(w_hbm.at[tok_v.at[pj, rj]],
                                 rows_v.at[kj], gsem.at[kj])
        return carry

    lax.fori_loop(0, GROUPS, group, 0, unroll=False)

    for k in range(NBUF):
        blk = ROWS_PER_TILE - NBUF + k
        _drain((blk % NBUF), blk)
    plsc.subcore_barrier()

    o = s * ROWS_OUT_PER_TILE
    pltpu.sync_copy(acc_sh.at[pl.ds(o, ROWS_OUT_PER_TILE)],
                    out_hbm.at[pl.ds(o, ROWS_OUT_PER_TILE),
                               pl.ds(c * EMBED, EMBED)])


@jax.jit
def kernel(added_tokens, added_batch_ids, deled_tokens, deled_batch_ids, W):
    tok = jnp.stack([added_tokens.astype(jnp.int32),
                     deled_tokens.astype(jnp.int32)]).reshape(2, TOK_ROWS, BLK)
    bid = jnp.stack([added_batch_ids.astype(jnp.int32),
                     deled_batch_ids.astype(jnp.int32)]).reshape(2, TOK_ROWS, BLK)
    zeros = jnp.zeros((ROWS_OUT_PER_TILE, EMBED), jnp.float32)

    mesh = plsc.VectorSubcoreMesh(core_axis_name="c", subcore_axis_name="s")
    run = pl.kernel(
        _body,
        out_type=jax.ShapeDtypeStruct((BATCH, 2 * EMBED), jnp.float32),
        mesh=mesh,
        scratch_types=[
            pltpu.VMEM((3, CHUNK_ROWS, BLK), jnp.int32),
            pltpu.VMEM((3, CHUNK_ROWS, BLK), jnp.int32),
            pltpu.VMEM((NBUF, BLK, EMBED), jnp.float32),
            pltpu.VMEM((NBUF, FL), jnp.int32),
            pltpu.VMEM_SHARED((ACC_ROWS, EMBED), jnp.float32),
            pltpu.SemaphoreType.DMA((NBUF,)),
            pltpu.SemaphoreType.DMA((NBUF,)),
            pltpu.SemaphoreType.DMA,
            pltpu.SemaphoreType.DMA,
        ],
    )
    return run(tok, bid, W, zeros)
